# baseline (speedup 1.0000x reference)
import functools

import jax
import jax.numpy as jnp
from jax import lax
from jax.experimental import pallas as pl
from jax.experimental.pallas import tpu as pltpu
from jax.experimental.pallas import tpu_sc as plsc

N = 3200000
NUM_ROWS = 100000
NP = 16
PER_P = N // NP
G = 4000
NG = PER_P // G
L = 16
SPG = 2 * G
RCHUNK = 4000

_mesh = plsc.VectorSubcoreMesh(core_axis_name="c", subcore_axis_name="s")


@functools.partial(
    pl.kernel,
    mesh=_mesh,
    compiler_params=pltpu.CompilerParams(needs_layout_passes=False),
    out_type=jax.ShapeDtypeStruct((2 * N,), jnp.int32),
    scratch_types=[
        pltpu.VMEM((NUM_ROWS,), jnp.int32),
        pltpu.VMEM((SPG,), jnp.int32),
        pltpu.VMEM((G,), jnp.int32),
        pltpu.VMEM((G,), jnp.int32),
        pltpu.VMEM_SHARED((8 * SPG,), jnp.int32),
        pltpu.SemaphoreType.DMA,
    ],
)
def _sc_gather(idx2_hbm, table2_hbm, out2_hbm,
               plane_v, stage_v, vals_v, oidx_v, merge_sp, sem):
    c = lax.axis_index("c")
    s = lax.axis_index("s")
    is_lo = s < jnp.int32(8)
    col = jnp.where(is_lo, jnp.int32(0), jnp.int32(1))
    q = lax.rem(s, jnp.int32(8))
    pair = c * jnp.int32(8) + q
    base = pair * jnp.int32(PER_P)
    iota2 = lax.iota(jnp.int32, L) * jnp.int32(2)

    def prow(k, carry):
        pltpu.sync_copy(table2_hbm.at[pl.ds(k * jnp.int32(2 * RCHUNK),
                                            2 * RCHUNK)], stage_v)
        pbase = k * jnp.int32(RCHUNK)
        pat = iota2 + col

        @plsc.parallel_loop(jnp.int32(0), jnp.int32(RCHUNK),
                            step=jnp.int32(L), unroll=8)
        def pbody(i):
            plane_v[pl.ds(pbase + i, L)] = plsc.load_gather(
                stage_v, [pat + i * jnp.int32(2)])

        return carry

    lax.fori_loop(jnp.int32(0), jnp.int32(NUM_ROWS // RCHUNK), prow, 0)

    obase = q * jnp.int32(SPG) + col

    @plsc.parallel_loop(jnp.int32(0), jnp.int32(G), step=jnp.int32(L),
                        unroll=8)
    def obody(j):
        oidx_v[pl.ds(j, L)] = iota2 + (obase + j * jnp.int32(2))

    def group(g, carry):
        off = base + g * jnp.int32(G)
        pltpu.sync_copy(idx2_hbm.at[pl.ds(off * jnp.int32(2), SPG)], stage_v)

        @plsc.parallel_loop(jnp.int32(0), jnp.int32(G), step=jnp.int32(L),
                            unroll=8)
        def gbody(i):
            ids = plsc.load_gather(stage_v, [iota2 + i * jnp.int32(2)])
            vals_v[pl.ds(i, L)] = plsc.load_gather(plane_v, [ids])

        pltpu.sync_copy(vals_v, merge_sp.at[oidx_v])
        plsc.subcore_barrier()

        @pl.when(is_lo)
        def _():
            pltpu.sync_copy(
                merge_sp.at[pl.ds(q * jnp.int32(SPG), SPG)], stage_v)
            pltpu.sync_copy(stage_v, out2_hbm.at[pl.ds(off * jnp.int32(2),
                                                       SPG)])

        plsc.subcore_barrier()
        return carry

    lax.fori_loop(jnp.int32(0), jnp.int32(NG), group, 0)


def kernel(n_id, last_update):
    idx2 = lax.bitcast_convert_type(n_id, jnp.int32).reshape(2 * N)
    table2 = lax.bitcast_convert_type(
        last_update, jnp.int32).reshape(2 * NUM_ROWS)
    out2 = _sc_gather(idx2, table2)
    return lax.bitcast_convert_type(out2.reshape(N, 2), jnp.int64)


# speedup vs baseline: 17.4562x; 17.4562x over previous
import functools

import jax
import jax.numpy as jnp
from jax import lax
from jax.experimental import pallas as pl
from jax.experimental.pallas import tpu as pltpu
from jax.experimental.pallas import tpu_sc as plsc

N = 3200000
NUM_ROWS = 100000
NT = 16
PER_T = N // NT
G = 8000
NG = PER_T // G
L = 16

_mesh = plsc.VectorSubcoreMesh(core_axis_name="c", subcore_axis_name="s")


@functools.partial(
    pl.kernel,
    mesh=_mesh,
    compiler_params=pltpu.CompilerParams(needs_layout_passes=False),
    out_type=(
        jax.ShapeDtypeStruct((N,), jnp.int32),
        jax.ShapeDtypeStruct((N,), jnp.int32),
    ),
    scratch_types=[
        pltpu.VMEM((NUM_ROWS,), jnp.int32),
        pltpu.VMEM((G,), jnp.int32),
        pltpu.VMEM((G,), jnp.int32),
        pltpu.SemaphoreType.DMA,
    ],
)
def _sc_gather(idx_hbm, lo_hbm, hi_hbm, out_lo_hbm, out_hi_hbm,
               plane_v, idx_v, vals_v, sem):
    wid = lax.axis_index("s") * 2 + lax.axis_index("c")
    is_lo = wid < jnp.int32(NT)
    slot = lax.rem(wid, jnp.int32(NT))
    base = slot * jnp.int32(PER_T)

    @pl.when(is_lo)
    def _():
        pltpu.sync_copy(lo_hbm, plane_v)

    @pl.when(jnp.logical_not(is_lo))
    def _():
        pltpu.sync_copy(hi_hbm, plane_v)

    def group(g, carry):
        off = base + g * jnp.int32(G)
        pltpu.sync_copy(idx_hbm.at[pl.ds(off, G)], idx_v)

        @plsc.parallel_loop(jnp.int32(0), jnp.int32(G), step=jnp.int32(L),
                            unroll=8)
        def gbody(i):
            ids = idx_v[pl.ds(i, L)]
            vals_v[pl.ds(i, L)] = plsc.load_gather(plane_v, [ids])

        @pl.when(is_lo)
        def _():
            pltpu.sync_copy(vals_v, out_lo_hbm.at[pl.ds(off, G)])

        @pl.when(jnp.logical_not(is_lo))
        def _():
            pltpu.sync_copy(vals_v, out_hi_hbm.at[pl.ds(off, G)])

        return carry

    lax.fori_loop(jnp.int32(0), jnp.int32(NG), group, 0)


def kernel(n_id, last_update):
    idx32 = n_id.astype(jnp.int32)
    table_lo = last_update.astype(jnp.int32)
    table_hi = (last_update >> 32).astype(jnp.int32)
    out_lo, out_hi = _sc_gather(idx32, table_lo, table_hi)
    return (out_hi.astype(jnp.int64) << 32) | (
        out_lo.astype(jnp.uint32).astype(jnp.int64))


# speedup vs baseline: 18.5552x; 1.0630x over previous
import functools

import jax
import jax.numpy as jnp
from jax import lax
from jax.experimental import pallas as pl
from jax.experimental.pallas import tpu as pltpu
from jax.experimental.pallas import tpu_sc as plsc

N = 3200000
NUM_ROWS = 100000
NT = 16
PER_T = N // NT
G = 5000
NG = PER_T // G
L = 16

_mesh = plsc.VectorSubcoreMesh(core_axis_name="c", subcore_axis_name="s")


@functools.partial(
    pl.kernel,
    mesh=_mesh,
    compiler_params=pltpu.CompilerParams(needs_layout_passes=False),
    out_type=(
        jax.ShapeDtypeStruct((N,), jnp.int32),
        jax.ShapeDtypeStruct((N,), jnp.int32),
    ),
    scratch_types=[
        pltpu.VMEM((NUM_ROWS,), jnp.int32),
        pltpu.VMEM((G,), jnp.uint32),
        pltpu.VMEM((G,), jnp.uint32),
        pltpu.VMEM((G,), jnp.int32),
        pltpu.VMEM((G,), jnp.int32),
        pltpu.SemaphoreType.DMA,
        pltpu.SemaphoreType.DMA,
    ],
)
def _sc_gather(idx_hbm, lo_hbm, hi_hbm, out_lo_hbm, out_hi_hbm,
               plane_v, idx0_v, idx1_v, vals0_v, vals1_v, sem_i, sem_o):
    wid = lax.axis_index("s") * 2 + lax.axis_index("c")
    is_lo = wid < jnp.int32(NT)
    slot = lax.rem(wid, jnp.int32(NT))
    base = slot * jnp.int32(PER_T)

    @pl.when(is_lo)
    def _():
        pltpu.sync_copy(lo_hbm, plane_v)

    @pl.when(jnp.logical_not(is_lo))
    def _():
        pltpu.sync_copy(hi_hbm, plane_v)

    def idx_copy(g, idx_b):
        return pltpu.make_async_copy(
            idx_hbm.at[pl.ds(base + g * jnp.int32(G), G)], idx_b, sem_i)

    def out_copy(g, vals_b):
        off = base + g * jnp.int32(G)

        @pl.when(is_lo)
        def _():
            pltpu.make_async_copy(
                vals_b, out_lo_hbm.at[pl.ds(off, G)], sem_o).start()

        @pl.when(jnp.logical_not(is_lo))
        def _():
            pltpu.make_async_copy(
                vals_b, out_hi_hbm.at[pl.ds(off, G)], sem_o).start()

    def out_wait(vals_b):
        pltpu.make_async_copy(
            vals_b, out_lo_hbm.at[pl.ds(base, G)], sem_o).wait()

    def gather(idx_b, vals_b):
        @plsc.parallel_loop(jnp.int32(0), jnp.int32(G), step=jnp.int32(L),
                            unroll=8)
        def gbody(i):
            ids = plsc.bitcast(idx_b[pl.ds(i, L)], jnp.int32)
            vals_b[pl.ds(i, L)] = plsc.load_gather(plane_v, [ids])

    idx_copy(jnp.int32(0), idx0_v).start()

    def pair(gg, carry):
        g0 = gg * jnp.int32(2)
        g1 = g0 + jnp.int32(1)

        idx_copy(g0, idx0_v).wait()
        idx_copy(g1, idx1_v).start()

        @pl.when(gg >= jnp.int32(1))
        def _():
            out_wait(vals0_v)

        gather(idx0_v, vals0_v)
        out_copy(g0, vals0_v)

        idx_copy(g1, idx1_v).wait()

        @pl.when(g1 + jnp.int32(1) < jnp.int32(NG))
        def _():
            idx_copy(g1 + jnp.int32(1), idx0_v).start()

        @pl.when(gg >= jnp.int32(1))
        def _():
            out_wait(vals1_v)

        gather(idx1_v, vals1_v)
        out_copy(g1, vals1_v)
        return carry

    lax.fori_loop(jnp.int32(0), jnp.int32(NG // 2), pair, 0)
    out_wait(vals0_v)
    out_wait(vals1_v)


def kernel(n_id, last_update):
    idx32 = n_id.astype(jnp.uint32)
    table_lo = last_update.astype(jnp.int32)
    table_hi = (last_update >> 32).astype(jnp.int32)
    out_lo, out_hi = _sc_gather(idx32, table_lo, table_hi)
    return (out_hi.astype(jnp.int64) << 32) | (
        out_lo.astype(jnp.uint32).astype(jnp.int64))
